# speedup vs baseline: 1.0282x; 1.0282x over previous
import jax
import jax.numpy as jnp
from jax.experimental import pallas as pl
from jax.experimental.pallas import tpu as pltpu

N_NODES = 10000
D = 128


def _mm_kernel(x_ref, w_ref, o_ref):
    o_ref[...] = jnp.dot(x_ref[...], w_ref[...], preferred_element_type=jnp.float32)


def _matmul(x, w):
    return pl.pallas_call(
        _mm_kernel,
        out_shape=jax.ShapeDtypeStruct((x.shape[0], w.shape[1]), jnp.float32),
    )(x, w)


def kernel(x, edge_index, edge_weight, W_gcn, b_gcn, W_mean, a_src_mean, a_dst_mean, b_mean, W_log, a_src_log, a_dst_log, b_log):
    src = edge_index[0]
    dst = edge_index[1]
    n = N_NODES

    loop = jnp.arange(n, dtype=src.dtype)
    src_f = jnp.concatenate([src, loop])
    dst_f = jnp.concatenate([dst, loop])
    w_f = jnp.concatenate([edge_weight, jnp.ones((n,), jnp.float32)])
    deg = jax.ops.segment_sum(w_f, dst_f, num_segments=n)
    dinv = jnp.where(deg > 0, jax.lax.rsqrt(deg), 0.0)
    norm = dinv[src_f] * w_f * dinv[dst_f]
    h1 = _matmul(x, W_gcn)
    msg = h1[src_f] * norm[:, None]
    h = jax.ops.segment_sum(msg, dst_f, num_segments=n) + b_gcn
    h = jax.nn.relu(h)

    def gat(h, W, a_s, a_d, b):
        hw = _matmul(h, W)
        alpha_s = (hw * a_s).sum(-1)
        alpha_d = (hw * a_d).sum(-1)
        e = alpha_s[src_f] + alpha_d[dst_f]
        e = jnp.where(e > 0, e, 0.2 * e)
        emax = jax.ops.segment_max(e, dst_f, num_segments=n)
        emax = jnp.where(jnp.isfinite(emax), emax, 0.0)
        ex = jnp.exp(e - emax[dst_f])
        denom = jax.ops.segment_sum(ex, dst_f, num_segments=n)
        alpha = ex / jnp.maximum(denom[dst_f], 1e-16)
        out = jax.ops.segment_sum(hw[src_f] * alpha[:, None], dst_f, num_segments=n)
        return out + b

    mean = gat(h, W_mean, a_src_mean, a_dst_mean, b_mean)
    logstd = gat(h, W_log, a_src_log, a_dst_log, b_log)
    return (mean, logstd)


# speedup vs baseline: 9.3945x; 9.1366x over previous
import jax
import jax.numpy as jnp
from jax import lax
from jax.experimental import pallas as pl
from jax.experimental.pallas import tpu as pltpu
from jax.experimental.pallas import tpu_sc as plsc

N = 10000
E = 320000
D = 128
NC, NS = 2, 16
NW = NC * NS
TPN = 313
NPAD = NW * TPN
FB = 2048
WB = E + FB + 16
CE = 128
CP = 2560
NEG = -3.0e38

_mesh = lambda: plsc.VectorSubcoreMesh(core_axis_name="c", subcore_axis_name="s")


def _wid():
    return lax.axis_index("s") * NC + lax.axis_index("c")


def _al(x):
    return pl.multiple_of(x, 8)


def _sload(ref, i):
    return ref[pl.ds(i, 16)][0]


def _zero_vmem(ref, nwords):
    def body(i, _):
        ref[pl.ds(i * 16, 16)] = jnp.zeros((16,), ref.dtype)
        return 0
    lax.fori_loop(0, nwords // 16, body, 0)


def _fill_vmem(ref, nwords, val):
    def body(i, _):
        ref[pl.ds(i * 16, 16)] = jnp.full((16,), val, ref.dtype)
        return 0
    lax.fori_loop(0, nwords // 16, body, 0)


def _plan_body(src_h, dst_h, ew_h,
               srcb_h, dstb_h, ewb_h, cnt_h, deg_h,
               src_c, dst_c, ew_c, st_src, st_dst, st_ew, deg_loc, cbuf):
    t = _wid()
    lo = t * TPN
    hi = lo + TPN
    row = t * WB

    _zero_vmem(st_src, FB + 16)
    _zero_vmem(st_dst, FB + 16)
    _zero_vmem(st_ew, FB + 16)
    _zero_vmem(deg_loc, 320)

    def chunk(ci, carry):
        off, goff = carry
        base = _al(ci * CP)
        pltpu.sync_copy(src_h.at[pl.ds(base, CP)], src_c)
        pltpu.sync_copy(dst_h.at[pl.ds(base, CP)], dst_c)
        pltpu.sync_copy(ew_h.at[pl.ds(base, CP)], ew_c)

        def vreg(j, carry2):
            off, goff = carry2
            dv = dst_c[pl.ds(j * 16, 16)]
            sv = src_c[pl.ds(j * 16, 16)]
            wv = ew_c[pl.ds(j * 16, 16)]
            m = (dv >= lo) & (dv < hi)
            cnt = plsc.all_reduce_population_count(m)[0]
            plsc.store_compressed(st_src.at[pl.ds(off, 16)], sv, mask=m)
            plsc.store_compressed(st_dst.at[pl.ds(off, 16)], dv, mask=m)
            plsc.store_compressed(st_ew.at[pl.ds(off, 16)], wv, mask=m)
            dloc = jnp.where(m, dv - lo, 0)
            plsc.addupdate_scatter(deg_loc, [dloc], jnp.where(m, wv, 0.0),
                                   mask=m)
            off = off + cnt
            full = off >= FB

            @pl.when(full)
            def _flush():
                pltpu.sync_copy(st_src.at[pl.ds(0, FB)],
                                srcb_h.at[pl.ds(_al(row + goff), FB)])
                pltpu.sync_copy(st_dst.at[pl.ds(0, FB)],
                                dstb_h.at[pl.ds(_al(row + goff), FB)])
                pltpu.sync_copy(st_ew.at[pl.ds(0, FB)],
                                ewb_h.at[pl.ds(_al(row + goff), FB)])
                for st in (st_src, st_dst, st_ew):
                    tail = st[pl.ds(FB, 16)]
                    st[pl.ds(0, 16)] = tail

            off = jnp.where(full, off - FB, off)
            goff = jnp.where(full, goff + FB, goff)
            return off, goff

        return lax.fori_loop(0, CP // 16, vreg, (off, goff))

    off, goff = lax.fori_loop(0, E // CP, chunk, (0, 0))

    pltpu.sync_copy(st_src.at[pl.ds(0, FB)], srcb_h.at[pl.ds(_al(row + goff), FB)])
    pltpu.sync_copy(st_dst.at[pl.ds(0, FB)], dstb_h.at[pl.ds(_al(row + goff), FB)])
    pltpu.sync_copy(st_ew.at[pl.ds(0, FB)], ewb_h.at[pl.ds(_al(row + goff), FB)])
    cbuf[...] = jnp.full((16,), goff + off, jnp.int32)
    pltpu.sync_copy(cbuf, cnt_h.at[pl.ds(_al(16 * t), 16)])
    pltpu.sync_copy(deg_loc, deg_h.at[pl.ds(_al(320 * t), 320)])


def _k_plan(src, dst, ew):
    f = pl.kernel(
        _plan_body,
        out_type=[
            jax.ShapeDtypeStruct((NW * WB,), jnp.int32),
            jax.ShapeDtypeStruct((NW * WB,), jnp.int32),
            jax.ShapeDtypeStruct((NW * WB,), jnp.float32),
            jax.ShapeDtypeStruct((NW * 16,), jnp.int32),
            jax.ShapeDtypeStruct((NW * 320,), jnp.float32),
        ],
        mesh=_mesh(),
        compiler_params=pltpu.CompilerParams(needs_layout_passes=False),
        scratch_types=[
            pltpu.VMEM((CP,), jnp.int32),
            pltpu.VMEM((CP,), jnp.int32),
            pltpu.VMEM((CP,), jnp.float32),
            pltpu.VMEM((FB + 16,), jnp.int32),
            pltpu.VMEM((FB + 16,), jnp.int32),
            pltpu.VMEM((FB + 16,), jnp.float32),
            pltpu.VMEM((320,), jnp.float32),
            pltpu.VMEM((16,), jnp.int32),
        ],
    )
    return f(src, dst, ew)


def _gcn_body(g_h, srcb_h, dstb_h, ewb_h, cnt_h,
              agg_h,
              src_idx, dst_c, ew_c, rows, acc, cbuf, sem):
    t = _wid()
    lo = t * TPN
    row = t * WB
    pltpu.sync_copy(cnt_h.at[pl.ds(_al(16 * t), 16)], cbuf)
    cnt = cbuf[...][0]

    _zero_vmem(acc, TPN * D)

    def chunk(ci, _):
        base = ci * CE
        pltpu.sync_copy(srcb_h.at[pl.ds(_al(row + base), CE)], src_idx)
        pltpu.sync_copy(dstb_h.at[pl.ds(_al(row + base), CE)], dst_c.at[pl.ds(0, CE)])
        pltpu.sync_copy(ewb_h.at[pl.ds(_al(row + base), CE)], ew_c.at[pl.ds(0, CE)])
        pltpu.async_copy(g_h.at[src_idx], rows, sem).wait()
        nin = jnp.minimum(cnt - base, CE)

        def edge(e, _):
            dl = _sload(dst_c, e) - lo
            w = _sload(ew_c, e)
            a = dl * D
            for k in range(8):
                acc[pl.ds(a + k * 16, 16)] = (
                    acc[pl.ds(a + k * 16, 16)] + w * rows[e, pl.ds(k * 16, 16)])
            return 0
        lax.fori_loop(0, nin, edge, 0)
        return 0

    nchunks = (cnt + CE - 1) // CE
    lax.fori_loop(0, nchunks, chunk, 0)
    pltpu.sync_copy(acc, agg_h.at[pl.ds(_al(lo * D), TPN * D)])


def _k_gcn(g, srcb, dstb, ewb, cnts):
    f = pl.kernel(
        _gcn_body,
        out_type=[jax.ShapeDtypeStruct((NPAD * D,), jnp.float32)],
        mesh=_mesh(),
        compiler_params=pltpu.CompilerParams(needs_layout_passes=False),
        scratch_types=[
            pltpu.VMEM((CE,), jnp.int32),
            pltpu.VMEM((CE + 16,), jnp.int32),
            pltpu.VMEM((CE + 16,), jnp.float32),
            pltpu.VMEM((CE, D), jnp.float32),
            pltpu.VMEM((TPN * D,), jnp.float32),
            pltpu.VMEM((16,), jnp.int32),
            pltpu.SemaphoreType.DMA,
        ],
    )
    return f(g, srcb, dstb, ewb, cnts)[0]


def _gat_body(asm_h, adm_h, asl_h, adl_h, srcb_h, dstb_h, cnt_h,
              emaxm_h, emaxl_h, denm_h, denl_h, exmb_h, exlb_h,
              asm_v, adm_v, asl_v, adl_v,
              emaxm, emaxl, denm, denl,
              src_c, dst_c, exm_c, exl_c, cbuf):
    t = _wid()
    lo = t * TPN
    row = t * WB
    pltpu.sync_copy(cnt_h.at[pl.ds(_al(16 * t), 16)], cbuf)
    cnt = cbuf[...][0]
    pltpu.sync_copy(asm_h, asm_v)
    pltpu.sync_copy(adm_h, adm_v)
    pltpu.sync_copy(asl_h, asl_v)
    pltpu.sync_copy(adl_h, adl_v)
    _fill_vmem(emaxm, 320, NEG)
    _fill_vmem(emaxl, 320, NEG)
    _zero_vmem(denm, 320)
    _zero_vmem(denl, 320)

    nchunks = (cnt + CE - 1) // CE

    def _edge_vals(sv, dv, m):
        svs = jnp.where(m, sv, 0)
        dvs = jnp.where(m, dv, 0)
        em = plsc.load_gather(asm_v, [svs]) + plsc.load_gather(adm_v, [dvs])
        el = plsc.load_gather(asl_v, [svs]) + plsc.load_gather(adl_v, [dvs])
        em = jnp.where(em > 0, em, 0.2 * em)
        el = jnp.where(el > 0, el, 0.2 * el)
        return em, el

    def _seg_max(ref, dloc, ev, m0):
        def again(mm):
            return jnp.any(mm)

        def body(mm):
            cur = plsc.load_gather(ref, [dloc], mask=mm)
            new = jnp.maximum(cur, ev)
            plsc.store_scatter(ref, [dloc], new, mask=mm)
            chk = plsc.load_gather(ref, [dloc], mask=mm)
            return mm & ~(chk >= ev)

        lax.while_loop(again, body, m0)

    def chunkA(ci, _):
        base = ci * CE
        pltpu.sync_copy(srcb_h.at[pl.ds(_al(row + base), CE)], src_c)
        pltpu.sync_copy(dstb_h.at[pl.ds(_al(row + base), CE)], dst_c)
        rem = cnt - base

        def vreg(j, _):
            sv = src_c[pl.ds(j * 16, 16)]
            dv = dst_c[pl.ds(j * 16, 16)]
            m = (lax.iota(jnp.int32, 16) + j * 16) < rem
            em, el = _edge_vals(sv, dv, m)
            dloc = jnp.where(m, dv - lo, 0)
            _seg_max(emaxm, dloc, em, m)
            _seg_max(emaxl, dloc, el, m)
            return 0
        lax.fori_loop(0, CE // 16, vreg, 0)
        return 0
    lax.fori_loop(0, nchunks, chunkA, 0)

    def chunkB(ci, _):
        base = ci * CE
        pltpu.sync_copy(srcb_h.at[pl.ds(_al(row + base), CE)], src_c)
        pltpu.sync_copy(dstb_h.at[pl.ds(_al(row + base), CE)], dst_c)
        rem = cnt - base

        def vreg(j, _):
            sv = src_c[pl.ds(j * 16, 16)]
            dv = dst_c[pl.ds(j * 16, 16)]
            m = (lax.iota(jnp.int32, 16) + j * 16) < rem
            em, el = _edge_vals(sv, dv, m)
            dloc = jnp.where(m, dv - lo, 0)
            exm = jnp.exp(em - plsc.load_gather(emaxm, [dloc]))
            exl = jnp.exp(el - plsc.load_gather(emaxl, [dloc]))
            exm = jnp.where(m, exm, 0.0)
            exl = jnp.where(m, exl, 0.0)
            plsc.addupdate_scatter(denm, [dloc], exm, mask=m)
            plsc.addupdate_scatter(denl, [dloc], exl, mask=m)
            exm_c[pl.ds(j * 16, 16)] = exm
            exl_c[pl.ds(j * 16, 16)] = exl
            return 0
        lax.fori_loop(0, CE // 16, vreg, 0)
        pltpu.sync_copy(exm_c, exmb_h.at[pl.ds(_al(row + base), CE)])
        pltpu.sync_copy(exl_c, exlb_h.at[pl.ds(_al(row + base), CE)])
        return 0
    lax.fori_loop(0, nchunks, chunkB, 0)

    pltpu.sync_copy(emaxm, emaxm_h.at[pl.ds(_al(320 * t), 320)])
    pltpu.sync_copy(emaxl, emaxl_h.at[pl.ds(_al(320 * t), 320)])
    pltpu.sync_copy(denm, denm_h.at[pl.ds(_al(320 * t), 320)])
    pltpu.sync_copy(denl, denl_h.at[pl.ds(_al(320 * t), 320)])


def _k_gat(asm, adm, asl, adl, srcb, dstb, cnts):
    f = pl.kernel(
        _gat_body,
        out_type=[
            jax.ShapeDtypeStruct((NW * 320,), jnp.float32),
            jax.ShapeDtypeStruct((NW * 320,), jnp.float32),
            jax.ShapeDtypeStruct((NW * 320,), jnp.float32),
            jax.ShapeDtypeStruct((NW * 320,), jnp.float32),
            jax.ShapeDtypeStruct((NW * WB,), jnp.float32),
            jax.ShapeDtypeStruct((NW * WB,), jnp.float32),
        ],
        mesh=_mesh(),
        compiler_params=pltpu.CompilerParams(needs_layout_passes=False),
        scratch_types=[
            pltpu.VMEM((N,), jnp.float32),
            pltpu.VMEM((N,), jnp.float32),
            pltpu.VMEM((N,), jnp.float32),
            pltpu.VMEM((N,), jnp.float32),
            pltpu.VMEM((320,), jnp.float32),
            pltpu.VMEM((320,), jnp.float32),
            pltpu.VMEM((320,), jnp.float32),
            pltpu.VMEM((320,), jnp.float32),
            pltpu.VMEM((CE,), jnp.int32),
            pltpu.VMEM((CE,), jnp.int32),
            pltpu.VMEM((CE,), jnp.float32),
            pltpu.VMEM((CE,), jnp.float32),
            pltpu.VMEM((16,), jnp.int32),
        ],
    )
    return f(asm, adm, asl, adl, srcb, dstb, cnts)


def _agg_body(h_h, srcb_h, dstb_h, exmb_h, exlb_h, cnt_h,
              nm_h, nl_h,
              src_idx, dst_c, exm_c, exl_c, rows, accm, accl, cbuf, sem):
    t = _wid()
    lo = t * TPN
    row = t * WB
    pltpu.sync_copy(cnt_h.at[pl.ds(_al(16 * t), 16)], cbuf)
    cnt = cbuf[...][0]

    _zero_vmem(accm, TPN * D)
    _zero_vmem(accl, TPN * D)

    def chunk(ci, _):
        base = ci * CE
        pltpu.sync_copy(srcb_h.at[pl.ds(_al(row + base), CE)], src_idx)
        pltpu.sync_copy(dstb_h.at[pl.ds(_al(row + base), CE)], dst_c.at[pl.ds(0, CE)])
        pltpu.sync_copy(exmb_h.at[pl.ds(_al(row + base), CE)], exm_c.at[pl.ds(0, CE)])
        pltpu.sync_copy(exlb_h.at[pl.ds(_al(row + base), CE)], exl_c.at[pl.ds(0, CE)])
        pltpu.async_copy(h_h.at[src_idx], rows, sem).wait()
        nin = jnp.minimum(cnt - base, CE)

        def edge(e, _):
            dl = _sload(dst_c, e) - lo
            wm = _sload(exm_c, e)
            wl = _sload(exl_c, e)
            a = dl * D
            for k in range(8):
                r = rows[e, pl.ds(k * 16, 16)]
                accm[pl.ds(a + k * 16, 16)] = accm[pl.ds(a + k * 16, 16)] + wm * r
                accl[pl.ds(a + k * 16, 16)] = accl[pl.ds(a + k * 16, 16)] + wl * r
            return 0
        lax.fori_loop(0, nin, edge, 0)
        return 0

    nchunks = (cnt + CE - 1) // CE
    lax.fori_loop(0, nchunks, chunk, 0)
    pltpu.sync_copy(accm, nm_h.at[pl.ds(_al(lo * D), TPN * D)])
    pltpu.sync_copy(accl, nl_h.at[pl.ds(_al(lo * D), TPN * D)])


def _k_agg(h, srcb, dstb, exmb, exlb, cnts):
    f = pl.kernel(
        _agg_body,
        out_type=[
            jax.ShapeDtypeStruct((NPAD * D,), jnp.float32),
            jax.ShapeDtypeStruct((NPAD * D,), jnp.float32),
        ],
        mesh=_mesh(),
        compiler_params=pltpu.CompilerParams(needs_layout_passes=False),
        scratch_types=[
            pltpu.VMEM((CE,), jnp.int32),
            pltpu.VMEM((CE + 16,), jnp.int32),
            pltpu.VMEM((CE + 16,), jnp.float32),
            pltpu.VMEM((CE + 16,), jnp.float32),
            pltpu.VMEM((CE, D), jnp.float32),
            pltpu.VMEM((TPN * D,), jnp.float32),
            pltpu.VMEM((TPN * D,), jnp.float32),
            pltpu.VMEM((16,), jnp.int32),
            pltpu.SemaphoreType.DMA,
        ],
    )
    return f(h, srcb, dstb, exmb, exlb, cnts)


def _t1_body(x_ref, w_ref, deg_ref, h1_ref, g_ref):
    h1 = jnp.dot(x_ref[...], w_ref[...], preferred_element_type=jnp.float32)
    dinv = lax.rsqrt(deg_ref[...] + 1.0)
    h1_ref[...] = h1
    g_ref[...] = dinv * h1


RB = 2000


def _rb(shape_cols):
    return pl.BlockSpec((RB, shape_cols), lambda i: (i, 0))


def _full(r, c):
    return pl.BlockSpec((r, c), lambda i: (0, 0))


def _t1(x, W_gcn, deg):
    return pl.pallas_call(
        _t1_body,
        grid=(N // RB,),
        in_specs=[_rb(D), _full(D, D), _rb(1)],
        out_specs=[_rb(D), _rb(D)],
        out_shape=[
            jax.ShapeDtypeStruct((N, D), jnp.float32),
            jax.ShapeDtypeStruct((N, D), jnp.float32),
        ],
    )(x, W_gcn, deg)


def _t3_body(agg_ref, h1_ref, deg_ref, b_ref, wm_ref, am_s_ref, am_d_ref,
             wl_ref, al_s_ref, al_d_ref,
             h_ref, sm_s_ref, sm_d_ref, sl_s_ref, sl_d_ref):
    dinv = lax.rsqrt(deg_ref[...] + 1.0)
    out1 = dinv * agg_ref[...] + (dinv * dinv) * h1_ref[...] + b_ref[...]
    h = jnp.maximum(out1, 0.0)
    h_ref[...] = h
    vm_s = jnp.dot(wm_ref[...], am_s_ref[...], preferred_element_type=jnp.float32)
    vm_d = jnp.dot(wm_ref[...], am_d_ref[...], preferred_element_type=jnp.float32)
    vl_s = jnp.dot(wl_ref[...], al_s_ref[...], preferred_element_type=jnp.float32)
    vl_d = jnp.dot(wl_ref[...], al_d_ref[...], preferred_element_type=jnp.float32)
    sm_s_ref[...] = jnp.dot(h, vm_s, preferred_element_type=jnp.float32)
    sm_d_ref[...] = jnp.dot(h, vm_d, preferred_element_type=jnp.float32)
    sl_s_ref[...] = jnp.dot(h, vl_s, preferred_element_type=jnp.float32)
    sl_d_ref[...] = jnp.dot(h, vl_d, preferred_element_type=jnp.float32)


def _t3(agg, h1, deg, b_gcn, W_mean, a_src_mean, a_dst_mean, W_log,
        a_src_log, a_dst_log):
    return pl.pallas_call(
        _t3_body,
        grid=(N // RB,),
        in_specs=[_rb(D), _rb(D), _rb(1), _full(1, D), _full(D, D),
                  _full(D, 1), _full(D, 1), _full(D, D), _full(D, 1),
                  _full(D, 1)],
        out_specs=[_rb(D), _rb(1), _rb(1), _rb(1), _rb(1)],
        out_shape=[
            jax.ShapeDtypeStruct((N, D), jnp.float32),
            jax.ShapeDtypeStruct((N, 1), jnp.float32),
            jax.ShapeDtypeStruct((N, 1), jnp.float32),
            jax.ShapeDtypeStruct((N, 1), jnp.float32),
            jax.ShapeDtypeStruct((N, 1), jnp.float32),
        ],
    )(agg, h1, deg, b_gcn.reshape(1, D), W_mean, a_src_mean.reshape(D, 1),
      a_dst_mean.reshape(D, 1), W_log, a_src_log.reshape(D, 1),
      a_dst_log.reshape(D, 1))


def _t4_body(nm_ref, nl_ref, h_ref,
             emaxm_ref, denm_ref, emaxl_ref, denl_ref,
             sm_s_ref, sm_d_ref, sl_s_ref, sl_d_ref,
             wm_ref, bm_ref, wl_ref, bl_ref,
             mean_ref, log_ref):
    h = h_ref[...]

    def head(n_agg, emax, den, s_s, s_d, w, b):
        e_self = s_s + s_d
        e_self = jnp.where(e_self > 0, e_self, 0.2 * e_self)
        emax_f = jnp.maximum(emax, e_self)
        r = jnp.exp(emax - emax_f)
        es = jnp.exp(e_self - emax_f)
        dn = r * den + es
        agg = (r * n_agg + es * h) / dn
        return jnp.dot(agg, w, preferred_element_type=jnp.float32) + b

    mean_ref[...] = head(nm_ref[...], emaxm_ref[...], denm_ref[...],
                         sm_s_ref[...], sm_d_ref[...], wm_ref[...], bm_ref[...])
    log_ref[...] = head(nl_ref[...], emaxl_ref[...], denl_ref[...],
                        sl_s_ref[...], sl_d_ref[...], wl_ref[...], bl_ref[...])


def _t4(nm, nl, h, emaxm, denm, emaxl, denl, sm_s, sm_d, sl_s, sl_d,
        W_mean, b_mean, W_log, b_log):
    return pl.pallas_call(
        _t4_body,
        grid=(N // RB,),
        in_specs=[_rb(D), _rb(D), _rb(D),
                  _rb(1), _rb(1), _rb(1), _rb(1),
                  _rb(1), _rb(1), _rb(1), _rb(1),
                  _full(D, D), _full(1, D), _full(D, D), _full(1, D)],
        out_specs=[_rb(D), _rb(D)],
        out_shape=[
            jax.ShapeDtypeStruct((N, D), jnp.float32),
            jax.ShapeDtypeStruct((N, D), jnp.float32),
        ],
    )(nm, nl, h, emaxm, denm, emaxl, denl, sm_s, sm_d, sl_s, sl_d,
      W_mean, b_mean.reshape(1, D), W_log, b_log.reshape(1, D))


def _unpad_nodes(flat):
    return flat.reshape(NW, 320)[:, :TPN].reshape(NPAD)[:N].reshape(N, 1)


def kernel(x, edge_index, edge_weight, W_gcn, b_gcn, W_mean, a_src_mean,
           a_dst_mean, b_mean, W_log, a_src_log, a_dst_log, b_log):
    src = edge_index[0].astype(jnp.int32)
    dst = edge_index[1].astype(jnp.int32)
    ew = edge_weight

    srcb, dstb, ewb, cnts, degp = _k_plan(src, dst, ew)
    deg = _unpad_nodes(degp)

    h1, g = _t1(x, W_gcn, deg)
    aggp = _k_gcn(g, srcb, dstb, ewb, cnts)
    agg = aggp.reshape(NPAD, D)[:N]

    h, sm_s, sm_d, sl_s, sl_d = _t3(agg, h1, deg, b_gcn, W_mean, a_src_mean,
                                    a_dst_mean, W_log, a_src_log, a_dst_log)

    emaxmp, emaxlp, denmp, denlp, exmb, exlb = _k_gat(
        sm_s.reshape(N), sm_d.reshape(N), sl_s.reshape(N), sl_d.reshape(N),
        srcb, dstb, cnts)

    nmp, nlp = _k_agg(h, srcb, dstb, exmb, exlb, cnts)

    mean, logstd = _t4(
        nmp.reshape(NPAD, D)[:N], nlp.reshape(NPAD, D)[:N], h,
        _unpad_nodes(emaxmp), _unpad_nodes(denmp),
        _unpad_nodes(emaxlp), _unpad_nodes(denlp),
        sm_s, sm_d, sl_s, sl_d, W_mean, b_mean, W_log, b_log)
    return (mean, logstd)


# speedup vs baseline: 12.6322x; 1.3446x over previous
import jax
import jax.numpy as jnp
from jax import lax
from jax.experimental import pallas as pl
from jax.experimental.pallas import tpu as pltpu
from jax.experimental.pallas import tpu_sc as plsc

N = 10000
E = 320000
D = 128
NC, NS = 2, 16
NW = NC * NS
TPN = 313
NPAD = NW * TPN
FB = 2048
WB = E + FB + 16
CE = 128
CG = 1024
CP = 6400
NEG = -3.0e38

_mesh = lambda: plsc.VectorSubcoreMesh(core_axis_name="c", subcore_axis_name="s")
_params = lambda: pltpu.CompilerParams(needs_layout_passes=False)


def _wid():
    return lax.axis_index("s") * NC + lax.axis_index("c")


def _al(x):
    return pl.multiple_of(x, 8)


def _sload(ref, i):
    return ref[pl.ds(i, 16)][0]


def _zero_vmem(ref, nwords):
    def body(i, _):
        ref[pl.ds(i * 16, 16)] = jnp.zeros((16,), ref.dtype)
        return 0
    lax.fori_loop(0, nwords // 16, body, 0)


def _fill_vmem(ref, nwords, val):
    def body(i, _):
        ref[pl.ds(i * 16, 16)] = jnp.full((16,), val, ref.dtype)
        return 0
    lax.fori_loop(0, nwords // 16, body, 0)


def _plan_body(src_h, dst_h, ew_h,
               srcb_h, dstb_h, ewb_h, cnt_h, deg_h,
               src0, src1, dst0, dst1, ew0, ew1,
               st_src, st_dst, st_ew, deg_loc, cbuf, sm0, sm1):
    t = _wid()
    lo = t * TPN
    hi = lo + TPN
    row = t * WB
    srcs, dsts, ews, sms = (src0, src1), (dst0, dst1), (ew0, ew1), (sm0, sm1)
    NCH = E // CP

    _zero_vmem(st_src, FB + 16)
    _zero_vmem(st_dst, FB + 16)
    _zero_vmem(st_ew, FB + 16)
    _zero_vmem(deg_loc, 320)

    def _issue_in(i, b):
        base = _al(i * CP)
        pltpu.async_copy(src_h.at[pl.ds(base, CP)], srcs[b], sms[b])
        pltpu.async_copy(dst_h.at[pl.ds(base, CP)], dsts[b], sms[b])
        pltpu.async_copy(ew_h.at[pl.ds(base, CP)], ews[b], sms[b])

    def _wait_in(i, b):
        base = _al(i * CP)
        pltpu.make_async_copy(src_h.at[pl.ds(base, CP)], srcs[b], sms[b]).wait()
        pltpu.make_async_copy(dst_h.at[pl.ds(base, CP)], dsts[b], sms[b]).wait()
        pltpu.make_async_copy(ew_h.at[pl.ds(base, CP)], ews[b], sms[b]).wait()

    _issue_in(0, 0)

    def super_it(i2, carry):
        for b in (0, 1):
            i = i2 * 2 + b
            nb = 1 - b

            @pl.when(i + 1 < NCH)
            def _():
                _issue_in(i + 1, nb)

            _wait_in(i, b)

            def vreg(j, carry2):
                off, goff = carry2
                dv = dsts[b][pl.ds(j * 16, 16)]
                sv = srcs[b][pl.ds(j * 16, 16)]
                wv = ews[b][pl.ds(j * 16, 16)]
                m = (dv >= lo) & (dv < hi)
                cnt = plsc.all_reduce_population_count(m)[0]
                plsc.store_compressed(st_src.at[pl.ds(off, 16)], sv, mask=m)
                plsc.store_compressed(st_dst.at[pl.ds(off, 16)], dv, mask=m)
                plsc.store_compressed(st_ew.at[pl.ds(off, 16)], wv, mask=m)
                dloc = jnp.where(m, dv - lo, 0)
                plsc.addupdate_scatter(deg_loc, [dloc], jnp.where(m, wv, 0.0),
                                       mask=m)
                off = off + cnt
                full = off >= FB

                @pl.when(full)
                def _flush():
                    pltpu.sync_copy(st_src.at[pl.ds(0, FB)],
                                    srcb_h.at[pl.ds(_al(row + goff), FB)])
                    pltpu.sync_copy(st_dst.at[pl.ds(0, FB)],
                                    dstb_h.at[pl.ds(_al(row + goff), FB)])
                    pltpu.sync_copy(st_ew.at[pl.ds(0, FB)],
                                    ewb_h.at[pl.ds(_al(row + goff), FB)])
                    for st in (st_src, st_dst, st_ew):
                        tail = st[pl.ds(FB, 16)]
                        st[pl.ds(0, 16)] = tail

                off = jnp.where(full, off - FB, off)
                goff = jnp.where(full, goff + FB, goff)
                return off, goff

            carry = lax.fori_loop(0, CP // 16, vreg, carry)
        return carry

    off, goff = lax.fori_loop(0, NCH // 2, super_it, (0, 0))

    pltpu.sync_copy(st_src.at[pl.ds(0, FB)], srcb_h.at[pl.ds(_al(row + goff), FB)])
    pltpu.sync_copy(st_dst.at[pl.ds(0, FB)], dstb_h.at[pl.ds(_al(row + goff), FB)])
    pltpu.sync_copy(st_ew.at[pl.ds(0, FB)], ewb_h.at[pl.ds(_al(row + goff), FB)])
    cbuf[...] = jnp.full((16,), goff + off, jnp.int32)
    pltpu.sync_copy(cbuf, cnt_h.at[pl.ds(_al(16 * t), 16)])
    pltpu.sync_copy(deg_loc, deg_h.at[pl.ds(_al(320 * t), 320)])


def _k_plan(src, dst, ew):
    f = pl.kernel(
        _plan_body,
        out_type=[
            jax.ShapeDtypeStruct((NW * WB,), jnp.int32),
            jax.ShapeDtypeStruct((NW * WB,), jnp.int32),
            jax.ShapeDtypeStruct((NW * WB,), jnp.float32),
            jax.ShapeDtypeStruct((NW * 16,), jnp.int32),
            jax.ShapeDtypeStruct((NW * 320,), jnp.float32),
        ],
        mesh=_mesh(),
        compiler_params=_params(),
        scratch_types=[
            pltpu.VMEM((CP,), jnp.int32),
            pltpu.VMEM((CP,), jnp.int32),
            pltpu.VMEM((CP,), jnp.int32),
            pltpu.VMEM((CP,), jnp.int32),
            pltpu.VMEM((CP,), jnp.float32),
            pltpu.VMEM((CP,), jnp.float32),
            pltpu.VMEM((FB + 16,), jnp.int32),
            pltpu.VMEM((FB + 16,), jnp.int32),
            pltpu.VMEM((FB + 16,), jnp.float32),
            pltpu.VMEM((320,), jnp.float32),
            pltpu.VMEM((16,), jnp.int32),
            pltpu.SemaphoreType.DMA,
            pltpu.SemaphoreType.DMA,
        ],
    )
    return f(src, dst, ew)


def _gcn_body(g_h, srcb_h, dstb_h, ewb_h, cnt_h,
              agg_h,
              src0, src1, dst0, dst1, ew0, ew1, rows0, rows1, acc, cbuf,
              sm0, sm1, sr0, sr1):
    t = _wid()
    lo = t * TPN
    row = t * WB
    pltpu.sync_copy(cnt_h.at[pl.ds(_al(16 * t), 16)], cbuf)
    cnt = cbuf[...][0]
    nchunks = (cnt + CE - 1) // CE
    srcs, dsts, ews = (src0, src1), (dst0, dst1), (ew0, ew1)
    rows, sms, srs = (rows0, rows1), (sm0, sm1), (sr0, sr1)

    _zero_vmem(acc, TPN * D)

    def _issue_meta(i, b):
        base = _al(row + i * CE)
        pltpu.async_copy(srcb_h.at[pl.ds(base, CE)], srcs[b], sms[b])
        pltpu.async_copy(dstb_h.at[pl.ds(base, CE)], dsts[b].at[pl.ds(0, CE)], sms[b])
        pltpu.async_copy(ewb_h.at[pl.ds(base, CE)], ews[b].at[pl.ds(0, CE)], sms[b])

    def _wait_meta(i, b):
        base = _al(row + i * CE)
        pltpu.make_async_copy(srcb_h.at[pl.ds(base, CE)], srcs[b], sms[b]).wait()
        pltpu.make_async_copy(dstb_h.at[pl.ds(base, CE)], dsts[b].at[pl.ds(0, CE)], sms[b]).wait()
        pltpu.make_async_copy(ewb_h.at[pl.ds(base, CE)], ews[b].at[pl.ds(0, CE)], sms[b]).wait()

    @pl.when(nchunks > 0)
    def _():
        _issue_meta(0, 0)
        _wait_meta(0, 0)
        pltpu.async_copy(g_h.at[srcs[0]], rows[0], srs[0])

    @pl.when(nchunks > 1)
    def _():
        _issue_meta(1, 1)

    def super_it(i2, _):
        for b in (0, 1):
            i = i2 * 2 + b
            nb = 1 - b

            @pl.when(i < nchunks)
            def _():
                @pl.when(i + 1 < nchunks)
                def _():
                    _wait_meta(i + 1, nb)
                    pltpu.async_copy(g_h.at[srcs[nb]], rows[nb], srs[nb])

                pltpu.make_async_copy(g_h.at[srcs[b]], rows[b], srs[b]).wait()
                base = i * CE
                nin = jnp.minimum(cnt - base, CE)

                def edge(e, _):
                    dl = _sload(dsts[b], e) - lo
                    w = _sload(ews[b], e)
                    a = dl * D
                    for k in range(8):
                        acc[pl.ds(a + k * 16, 16)] = (
                            acc[pl.ds(a + k * 16, 16)]
                            + w * rows[b][e, pl.ds(k * 16, 16)])
                    return 0
                lax.fori_loop(0, nin, edge, 0)

                @pl.when(i + 2 < nchunks)
                def _():
                    _issue_meta(i + 2, b)
        return 0

    lax.fori_loop(0, (nchunks + 1) // 2, super_it, 0)
    pltpu.sync_copy(acc, agg_h.at[pl.ds(_al(lo * D), TPN * D)])


def _k_gcn(g, srcb, dstb, ewb, cnts):
    f = pl.kernel(
        _gcn_body,
        out_type=[jax.ShapeDtypeStruct((NPAD * D,), jnp.float32)],
        mesh=_mesh(),
        compiler_params=_params(),
        scratch_types=[
            pltpu.VMEM((CE,), jnp.int32),
            pltpu.VMEM((CE,), jnp.int32),
            pltpu.VMEM((CE + 16,), jnp.int32),
            pltpu.VMEM((CE + 16,), jnp.int32),
            pltpu.VMEM((CE + 16,), jnp.float32),
            pltpu.VMEM((CE + 16,), jnp.float32),
            pltpu.VMEM((CE, D), jnp.float32),
            pltpu.VMEM((CE, D), jnp.float32),
            pltpu.VMEM((TPN * D,), jnp.float32),
            pltpu.VMEM((16,), jnp.int32),
            pltpu.SemaphoreType.DMA,
            pltpu.SemaphoreType.DMA,
            pltpu.SemaphoreType.DMA,
            pltpu.SemaphoreType.DMA,
        ],
    )
    return f(g, srcb, dstb, ewb, cnts)[0]


def _gat_body(asm_h, adm_h, asl_h, adl_h, srcb_h, dstb_h, cnt_h,
              emaxm_h, emaxl_h, denm_h, denl_h, exmb_h, exlb_h,
              asm_v, adm_v, asl_v, adl_v,
              emaxm, emaxl, denm, denl,
              src_c, dst_c, exm_c, exl_c, cbuf):
    t = _wid()
    lo = t * TPN
    row = t * WB
    pltpu.sync_copy(cnt_h.at[pl.ds(_al(16 * t), 16)], cbuf)
    cnt = cbuf[...][0]
    pltpu.sync_copy(asm_h, asm_v)
    pltpu.sync_copy(adm_h, adm_v)
    pltpu.sync_copy(asl_h, asl_v)
    pltpu.sync_copy(adl_h, adl_v)
    _fill_vmem(emaxm, 320, NEG)
    _fill_vmem(emaxl, 320, NEG)
    _zero_vmem(denm, 320)
    _zero_vmem(denl, 320)

    nchunks = (cnt + CG - 1) // CG

    def _edge_vals(sv, dv, m):
        svs = jnp.where(m, sv, 0)
        dvs = jnp.where(m, dv, 0)
        em = plsc.load_gather(asm_v, [svs]) + plsc.load_gather(adm_v, [dvs])
        el = plsc.load_gather(asl_v, [svs]) + plsc.load_gather(adl_v, [dvs])
        em = jnp.where(em > 0, em, 0.2 * em)
        el = jnp.where(el > 0, el, 0.2 * el)
        return em, el

    def _seg_max(ref, dloc, ev, m0):
        def again(mm):
            return jnp.any(mm)

        def body(mm):
            cur = plsc.load_gather(ref, [dloc], mask=mm)
            new = jnp.maximum(cur, ev)
            plsc.store_scatter(ref, [dloc], new, mask=mm)
            chk = plsc.load_gather(ref, [dloc], mask=mm)
            return mm & ~(chk >= ev)

        lax.while_loop(again, body, m0)

    def chunkA(ci, _):
        base = ci * CG
        pltpu.sync_copy(srcb_h.at[pl.ds(_al(row + base), CG)], src_c)
        pltpu.sync_copy(dstb_h.at[pl.ds(_al(row + base), CG)], dst_c)
        rem = cnt - base

        def vreg(j, _):
            sv = src_c[pl.ds(j * 16, 16)]
            dv = dst_c[pl.ds(j * 16, 16)]
            m = (lax.iota(jnp.int32, 16) + j * 16) < rem
            em, el = _edge_vals(sv, dv, m)
            dloc = jnp.where(m, dv - lo, 0)
            _seg_max(emaxm, dloc, em, m)
            _seg_max(emaxl, dloc, el, m)
            return 0
        lax.fori_loop(0, CG // 16, vreg, 0)
        return 0
    lax.fori_loop(0, nchunks, chunkA, 0)

    def chunkB(ci, _):
        base = ci * CG
        pltpu.sync_copy(srcb_h.at[pl.ds(_al(row + base), CG)], src_c)
        pltpu.sync_copy(dstb_h.at[pl.ds(_al(row + base), CG)], dst_c)
        rem = cnt - base

        def vreg(j, _):
            sv = src_c[pl.ds(j * 16, 16)]
            dv = dst_c[pl.ds(j * 16, 16)]
            m = (lax.iota(jnp.int32, 16) + j * 16) < rem
            em, el = _edge_vals(sv, dv, m)
            dloc = jnp.where(m, dv - lo, 0)
            exm = jnp.exp(em - plsc.load_gather(emaxm, [dloc]))
            exl = jnp.exp(el - plsc.load_gather(emaxl, [dloc]))
            exm = jnp.where(m, exm, 0.0)
            exl = jnp.where(m, exl, 0.0)
            plsc.addupdate_scatter(denm, [dloc], exm, mask=m)
            plsc.addupdate_scatter(denl, [dloc], exl, mask=m)
            exm_c[pl.ds(j * 16, 16)] = exm
            exl_c[pl.ds(j * 16, 16)] = exl
            return 0
        lax.fori_loop(0, CG // 16, vreg, 0)
        pltpu.sync_copy(exm_c, exmb_h.at[pl.ds(_al(row + base), CG)])
        pltpu.sync_copy(exl_c, exlb_h.at[pl.ds(_al(row + base), CG)])
        return 0
    lax.fori_loop(0, nchunks, chunkB, 0)

    pltpu.sync_copy(emaxm, emaxm_h.at[pl.ds(_al(320 * t), 320)])
    pltpu.sync_copy(emaxl, emaxl_h.at[pl.ds(_al(320 * t), 320)])
    pltpu.sync_copy(denm, denm_h.at[pl.ds(_al(320 * t), 320)])
    pltpu.sync_copy(denl, denl_h.at[pl.ds(_al(320 * t), 320)])


def _k_gat(asm, adm, asl, adl, srcb, dstb, cnts):
    f = pl.kernel(
        _gat_body,
        out_type=[
            jax.ShapeDtypeStruct((NW * 320,), jnp.float32),
            jax.ShapeDtypeStruct((NW * 320,), jnp.float32),
            jax.ShapeDtypeStruct((NW * 320,), jnp.float32),
            jax.ShapeDtypeStruct((NW * 320,), jnp.float32),
            jax.ShapeDtypeStruct((NW * WB,), jnp.float32),
            jax.ShapeDtypeStruct((NW * WB,), jnp.float32),
        ],
        mesh=_mesh(),
        compiler_params=_params(),
        scratch_types=[
            pltpu.VMEM((N,), jnp.float32),
            pltpu.VMEM((N,), jnp.float32),
            pltpu.VMEM((N,), jnp.float32),
            pltpu.VMEM((N,), jnp.float32),
            pltpu.VMEM((320,), jnp.float32),
            pltpu.VMEM((320,), jnp.float32),
            pltpu.VMEM((320,), jnp.float32),
            pltpu.VMEM((320,), jnp.float32),
            pltpu.VMEM((CG,), jnp.int32),
            pltpu.VMEM((CG,), jnp.int32),
            pltpu.VMEM((CG,), jnp.float32),
            pltpu.VMEM((CG,), jnp.float32),
            pltpu.VMEM((16,), jnp.int32),
        ],
    )
    return f(asm, adm, asl, adl, srcb, dstb, cnts)


def _agg_body(h_h, srcb_h, dstb_h, exmb_h, exlb_h, cnt_h,
              nm_h, nl_h,
              src0, src1, dst0, dst1, xm0, xm1, xl0, xl1, rows0, rows1,
              accm, accl, cbuf, sm0, sm1, sr0, sr1):
    t = _wid()
    lo = t * TPN
    row = t * WB
    pltpu.sync_copy(cnt_h.at[pl.ds(_al(16 * t), 16)], cbuf)
    cnt = cbuf[...][0]
    nchunks = (cnt + CE - 1) // CE
    srcs, dsts = (src0, src1), (dst0, dst1)
    xms, xls = (xm0, xm1), (xl0, xl1)
    rows, sms, srs = (rows0, rows1), (sm0, sm1), (sr0, sr1)

    _zero_vmem(accm, TPN * D)
    _zero_vmem(accl, TPN * D)

    def _issue_meta(i, b):
        base = _al(row + i * CE)
        pltpu.async_copy(srcb_h.at[pl.ds(base, CE)], srcs[b], sms[b])
        pltpu.async_copy(dstb_h.at[pl.ds(base, CE)], dsts[b].at[pl.ds(0, CE)], sms[b])
        pltpu.async_copy(exmb_h.at[pl.ds(base, CE)], xms[b].at[pl.ds(0, CE)], sms[b])
        pltpu.async_copy(exlb_h.at[pl.ds(base, CE)], xls[b].at[pl.ds(0, CE)], sms[b])

    def _wait_meta(i, b):
        base = _al(row + i * CE)
        pltpu.make_async_copy(srcb_h.at[pl.ds(base, CE)], srcs[b], sms[b]).wait()
        pltpu.make_async_copy(dstb_h.at[pl.ds(base, CE)], dsts[b].at[pl.ds(0, CE)], sms[b]).wait()
        pltpu.make_async_copy(exmb_h.at[pl.ds(base, CE)], xms[b].at[pl.ds(0, CE)], sms[b]).wait()
        pltpu.make_async_copy(exlb_h.at[pl.ds(base, CE)], xls[b].at[pl.ds(0, CE)], sms[b]).wait()

    @pl.when(nchunks > 0)
    def _():
        _issue_meta(0, 0)
        _wait_meta(0, 0)
        pltpu.async_copy(h_h.at[srcs[0]], rows[0], srs[0])

    @pl.when(nchunks > 1)
    def _():
        _issue_meta(1, 1)

    def super_it(i2, _):
        for b in (0, 1):
            i = i2 * 2 + b
            nb = 1 - b

            @pl.when(i < nchunks)
            def _():
                @pl.when(i + 1 < nchunks)
                def _():
                    _wait_meta(i + 1, nb)
                    pltpu.async_copy(h_h.at[srcs[nb]], rows[nb], srs[nb])

                pltpu.make_async_copy(h_h.at[srcs[b]], rows[b], srs[b]).wait()
                base = i * CE
                nin = jnp.minimum(cnt - base, CE)

                def edge(e, _):
                    dl = _sload(dsts[b], e) - lo
                    wm = _sload(xms[b], e)
                    wl = _sload(xls[b], e)
                    a = dl * D
                    for k in range(8):
                        r = rows[b][e, pl.ds(k * 16, 16)]
                        accm[pl.ds(a + k * 16, 16)] = (
                            accm[pl.ds(a + k * 16, 16)] + wm * r)
                        accl[pl.ds(a + k * 16, 16)] = (
                            accl[pl.ds(a + k * 16, 16)] + wl * r)
                    return 0
                lax.fori_loop(0, nin, edge, 0)

                @pl.when(i + 2 < nchunks)
                def _():
                    _issue_meta(i + 2, b)
        return 0

    lax.fori_loop(0, (nchunks + 1) // 2, super_it, 0)
    pltpu.sync_copy(accm, nm_h.at[pl.ds(_al(lo * D), TPN * D)])
    pltpu.sync_copy(accl, nl_h.at[pl.ds(_al(lo * D), TPN * D)])


def _k_agg(h, srcb, dstb, exmb, exlb, cnts):
    f = pl.kernel(
        _agg_body,
        out_type=[
            jax.ShapeDtypeStruct((NPAD * D,), jnp.float32),
            jax.ShapeDtypeStruct((NPAD * D,), jnp.float32),
        ],
        mesh=_mesh(),
        compiler_params=_params(),
        scratch_types=[
            pltpu.VMEM((CE,), jnp.int32),
            pltpu.VMEM((CE,), jnp.int32),
            pltpu.VMEM((CE + 16,), jnp.int32),
            pltpu.VMEM((CE + 16,), jnp.int32),
            pltpu.VMEM((CE + 16,), jnp.float32),
            pltpu.VMEM((CE + 16,), jnp.float32),
            pltpu.VMEM((CE + 16,), jnp.float32),
            pltpu.VMEM((CE + 16,), jnp.float32),
            pltpu.VMEM((CE, D), jnp.float32),
            pltpu.VMEM((CE, D), jnp.float32),
            pltpu.VMEM((TPN * D,), jnp.float32),
            pltpu.VMEM((TPN * D,), jnp.float32),
            pltpu.VMEM((16,), jnp.int32),
            pltpu.SemaphoreType.DMA,
            pltpu.SemaphoreType.DMA,
            pltpu.SemaphoreType.DMA,
            pltpu.SemaphoreType.DMA,
        ],
    )
    return f(h, srcb, dstb, exmb, exlb, cnts)


RB = 2000


def _rb(shape_cols):
    return pl.BlockSpec((RB, shape_cols), lambda i: (i, 0))


def _full(r, c):
    return pl.BlockSpec((r, c), lambda i: (0, 0))


def _t1_body(x_ref, w_ref, deg_ref, h1_ref, g_ref):
    h1 = jnp.dot(x_ref[...], w_ref[...], preferred_element_type=jnp.float32)
    dinv = lax.rsqrt(deg_ref[...] + 1.0)
    h1_ref[...] = h1
    g_ref[...] = dinv * h1


def _t1(x, W_gcn, deg):
    return pl.pallas_call(
        _t1_body,
        grid=(N // RB,),
        in_specs=[_rb(D), _full(D, D), _rb(1)],
        out_specs=[_rb(D), _rb(D)],
        out_shape=[
            jax.ShapeDtypeStruct((N, D), jnp.float32),
            jax.ShapeDtypeStruct((N, D), jnp.float32),
        ],
    )(x, W_gcn, deg)


def _t3_body(agg_ref, h1_ref, deg_ref, b_ref, wm_ref, am_s_ref, am_d_ref,
             wl_ref, al_s_ref, al_d_ref,
             h_ref, sm_s_ref, sm_d_ref, sl_s_ref, sl_d_ref):
    dinv = lax.rsqrt(deg_ref[...] + 1.0)
    out1 = dinv * agg_ref[...] + (dinv * dinv) * h1_ref[...] + b_ref[...]
    h = jnp.maximum(out1, 0.0)
    h_ref[...] = h
    vm_s = jnp.dot(wm_ref[...], am_s_ref[...], preferred_element_type=jnp.float32)
    vm_d = jnp.dot(wm_ref[...], am_d_ref[...], preferred_element_type=jnp.float32)
    vl_s = jnp.dot(wl_ref[...], al_s_ref[...], preferred_element_type=jnp.float32)
    vl_d = jnp.dot(wl_ref[...], al_d_ref[...], preferred_element_type=jnp.float32)
    sm_s_ref[...] = jnp.dot(h, vm_s, preferred_element_type=jnp.float32)
    sm_d_ref[...] = jnp.dot(h, vm_d, preferred_element_type=jnp.float32)
    sl_s_ref[...] = jnp.dot(h, vl_s, preferred_element_type=jnp.float32)
    sl_d_ref[...] = jnp.dot(h, vl_d, preferred_element_type=jnp.float32)


def _t3(agg, h1, deg, b_gcn, W_mean, a_src_mean, a_dst_mean, W_log,
        a_src_log, a_dst_log):
    return pl.pallas_call(
        _t3_body,
        grid=(N // RB,),
        in_specs=[_rb(D), _rb(D), _rb(1), _full(1, D), _full(D, D),
                  _full(D, 1), _full(D, 1), _full(D, D), _full(D, 1),
                  _full(D, 1)],
        out_specs=[_rb(D), _rb(1), _rb(1), _rb(1), _rb(1)],
        out_shape=[
            jax.ShapeDtypeStruct((N, D), jnp.float32),
            jax.ShapeDtypeStruct((N, 1), jnp.float32),
            jax.ShapeDtypeStruct((N, 1), jnp.float32),
            jax.ShapeDtypeStruct((N, 1), jnp.float32),
            jax.ShapeDtypeStruct((N, 1), jnp.float32),
        ],
    )(agg, h1, deg, b_gcn.reshape(1, D), W_mean, a_src_mean.reshape(D, 1),
      a_dst_mean.reshape(D, 1), W_log, a_src_log.reshape(D, 1),
      a_dst_log.reshape(D, 1))


def _t4_body(nm_ref, nl_ref, h_ref,
             emaxm_ref, denm_ref, emaxl_ref, denl_ref,
             sm_s_ref, sm_d_ref, sl_s_ref, sl_d_ref,
             wm_ref, bm_ref, wl_ref, bl_ref,
             mean_ref, log_ref):
    h = h_ref[...]

    def head(n_agg, emax, den, s_s, s_d, w, b):
        e_self = s_s + s_d
        e_self = jnp.where(e_self > 0, e_self, 0.2 * e_self)
        emax_f = jnp.maximum(emax, e_self)
        r = jnp.exp(emax - emax_f)
        es = jnp.exp(e_self - emax_f)
        dn = r * den + es
        agg = (r * n_agg + es * h) / dn
        return jnp.dot(agg, w, preferred_element_type=jnp.float32) + b

    mean_ref[...] = head(nm_ref[...], emaxm_ref[...], denm_ref[...],
                         sm_s_ref[...], sm_d_ref[...], wm_ref[...], bm_ref[...])
    log_ref[...] = head(nl_ref[...], emaxl_ref[...], denl_ref[...],
                        sl_s_ref[...], sl_d_ref[...], wl_ref[...], bl_ref[...])


def _t4(nm, nl, h, emaxm, denm, emaxl, denl, sm_s, sm_d, sl_s, sl_d,
        W_mean, b_mean, W_log, b_log):
    return pl.pallas_call(
        _t4_body,
        grid=(N // RB,),
        in_specs=[_rb(D), _rb(D), _rb(D),
                  _rb(1), _rb(1), _rb(1), _rb(1),
                  _rb(1), _rb(1), _rb(1), _rb(1),
                  _full(D, D), _full(1, D), _full(D, D), _full(1, D)],
        out_specs=[_rb(D), _rb(D)],
        out_shape=[
            jax.ShapeDtypeStruct((N, D), jnp.float32),
            jax.ShapeDtypeStruct((N, D), jnp.float32),
        ],
    )(nm, nl, h, emaxm, denm, emaxl, denl, sm_s, sm_d, sl_s, sl_d,
      W_mean, b_mean.reshape(1, D), W_log, b_log.reshape(1, D))


def _unpad_nodes(flat):
    return flat.reshape(NW, 320)[:, :TPN].reshape(NPAD)[:N].reshape(N, 1)


def kernel(x, edge_index, edge_weight, W_gcn, b_gcn, W_mean, a_src_mean,
           a_dst_mean, b_mean, W_log, a_src_log, a_dst_log, b_log):
    src = edge_index[0].astype(jnp.int32)
    dst = edge_index[1].astype(jnp.int32)
    ew = edge_weight

    srcb, dstb, ewb, cnts, degp = _k_plan(src, dst, ew)
    deg = _unpad_nodes(degp)

    h1, g = _t1(x, W_gcn, deg)
    aggp = _k_gcn(g, srcb, dstb, ewb, cnts)
    agg = aggp.reshape(NPAD, D)[:N]

    h, sm_s, sm_d, sl_s, sl_d = _t3(agg, h1, deg, b_gcn, W_mean, a_src_mean,
                                    a_dst_mean, W_log, a_src_log, a_dst_log)

    emaxmp, emaxlp, denmp, denlp, exmb, exlb = _k_gat(
        sm_s.reshape(N), sm_d.reshape(N), sl_s.reshape(N), sl_d.reshape(N),
        srcb, dstb, cnts)

    nmp, nlp = _k_agg(h, srcb, dstb, exmb, exlb, cnts)

    mean, logstd = _t4(
        nmp.reshape(NPAD, D)[:N], nlp.reshape(NPAD, D)[:N], h,
        _unpad_nodes(emaxmp), _unpad_nodes(denmp),
        _unpad_nodes(emaxlp), _unpad_nodes(denlp),
        sm_s, sm_d, sl_s, sl_d, W_mean, b_mean, W_log, b_log)
    return (mean, logstd)


# speedup vs baseline: 14.1430x; 1.1196x over previous
import jax
import jax.numpy as jnp
from jax import lax
from jax.experimental import pallas as pl
from jax.experimental.pallas import tpu as pltpu
from jax.experimental.pallas import tpu_sc as plsc

N = 10000
E = 320000
D = 128
NC, NS = 2, 16
NW = NC * NS
TPN = 313
NPAD = NW * TPN
FB = 2048
WB = E + FB + 16
CE = 128
CG = 1024
CP = 6400
NEG = -3.0e38

_mesh = lambda: plsc.VectorSubcoreMesh(core_axis_name="c", subcore_axis_name="s")
_params = lambda: pltpu.CompilerParams(needs_layout_passes=False)


def _wid():
    return lax.axis_index("s") * NC + lax.axis_index("c")


def _al(x):
    return pl.multiple_of(x, 8)


def _sload(ref, i):
    return ref[pl.ds(i, 16)][0]


def _zero_vmem(ref, nwords):
    def body(i, _):
        ref[pl.ds(i * 16, 16)] = jnp.zeros((16,), ref.dtype)
        return 0
    lax.fori_loop(0, nwords // 16, body, 0)


def _fill_vmem(ref, nwords, val):
    def body(i, _):
        ref[pl.ds(i * 16, 16)] = jnp.full((16,), val, ref.dtype)
        return 0
    lax.fori_loop(0, nwords // 16, body, 0)


def _plan_body(src_h, dst_h, ew_h,
               srcb_h, dstb_h, ewb_h, cnt_h, deg_h,
               src0, src1, dst0, dst1, ew0, ew1,
               st_src, st_dst, st_ew, deg_loc, cbuf, sm0, sm1):
    t = _wid()
    lo = t * TPN
    hi = lo + TPN
    row = t * WB
    srcs, dsts, ews, sms = (src0, src1), (dst0, dst1), (ew0, ew1), (sm0, sm1)
    NCH = E // CP

    _zero_vmem(st_src, FB + 16)
    _zero_vmem(st_dst, FB + 16)
    _zero_vmem(st_ew, FB + 16)
    _zero_vmem(deg_loc, 320)

    def _issue_in(i, b):
        base = _al(i * CP)
        pltpu.async_copy(src_h.at[pl.ds(base, CP)], srcs[b], sms[b])
        pltpu.async_copy(dst_h.at[pl.ds(base, CP)], dsts[b], sms[b])
        pltpu.async_copy(ew_h.at[pl.ds(base, CP)], ews[b], sms[b])

    def _wait_in(i, b):
        base = _al(i * CP)
        pltpu.make_async_copy(src_h.at[pl.ds(base, CP)], srcs[b], sms[b]).wait()
        pltpu.make_async_copy(dst_h.at[pl.ds(base, CP)], dsts[b], sms[b]).wait()
        pltpu.make_async_copy(ew_h.at[pl.ds(base, CP)], ews[b], sms[b]).wait()

    _issue_in(0, 0)

    def super_it(i2, carry):
        for b in (0, 1):
            i = i2 * 2 + b
            nb = 1 - b

            @pl.when(i + 1 < NCH)
            def _():
                _issue_in(i + 1, nb)

            _wait_in(i, b)

            def vreg(j, carry2):
                off, goff = carry2
                dv = dsts[b][pl.ds(j * 16, 16)]
                sv = srcs[b][pl.ds(j * 16, 16)]
                wv = ews[b][pl.ds(j * 16, 16)]
                m = (dv >= lo) & (dv < hi)
                cnt = plsc.all_reduce_population_count(m)[0]
                plsc.store_compressed(st_src.at[pl.ds(off, 16)], sv, mask=m)
                plsc.store_compressed(st_dst.at[pl.ds(off, 16)], dv, mask=m)
                plsc.store_compressed(st_ew.at[pl.ds(off, 16)], wv, mask=m)
                dloc = jnp.where(m, dv - lo, 0)
                plsc.addupdate_scatter(deg_loc, [dloc], jnp.where(m, wv, 0.0),
                                       mask=m)
                off = off + cnt
                full = off >= FB

                @pl.when(full)
                def _flush():
                    pltpu.sync_copy(st_src.at[pl.ds(0, FB)],
                                    srcb_h.at[pl.ds(_al(row + goff), FB)])
                    pltpu.sync_copy(st_dst.at[pl.ds(0, FB)],
                                    dstb_h.at[pl.ds(_al(row + goff), FB)])
                    pltpu.sync_copy(st_ew.at[pl.ds(0, FB)],
                                    ewb_h.at[pl.ds(_al(row + goff), FB)])
                    for st in (st_src, st_dst, st_ew):
                        tail = st[pl.ds(FB, 16)]
                        st[pl.ds(0, 16)] = tail

                off = jnp.where(full, off - FB, off)
                goff = jnp.where(full, goff + FB, goff)
                return off, goff

            carry = lax.fori_loop(0, CP // 16, vreg, carry)
        return carry

    off, goff = lax.fori_loop(0, NCH // 2, super_it, (0, 0))

    pltpu.sync_copy(st_src.at[pl.ds(0, FB)], srcb_h.at[pl.ds(_al(row + goff), FB)])
    pltpu.sync_copy(st_dst.at[pl.ds(0, FB)], dstb_h.at[pl.ds(_al(row + goff), FB)])
    pltpu.sync_copy(st_ew.at[pl.ds(0, FB)], ewb_h.at[pl.ds(_al(row + goff), FB)])
    cbuf[...] = jnp.full((16,), goff + off, jnp.int32)
    pltpu.sync_copy(cbuf, cnt_h.at[pl.ds(_al(16 * t), 16)])
    pltpu.sync_copy(deg_loc, deg_h.at[pl.ds(_al(320 * t), 320)])


def _k_plan(src, dst, ew):
    f = pl.kernel(
        _plan_body,
        out_type=[
            jax.ShapeDtypeStruct((NW * WB,), jnp.int32),
            jax.ShapeDtypeStruct((NW * WB,), jnp.int32),
            jax.ShapeDtypeStruct((NW * WB,), jnp.float32),
            jax.ShapeDtypeStruct((NW * 16,), jnp.int32),
            jax.ShapeDtypeStruct((NW * 320,), jnp.float32),
        ],
        mesh=_mesh(),
        compiler_params=_params(),
        scratch_types=[
            pltpu.VMEM((CP,), jnp.int32),
            pltpu.VMEM((CP,), jnp.int32),
            pltpu.VMEM((CP,), jnp.int32),
            pltpu.VMEM((CP,), jnp.int32),
            pltpu.VMEM((CP,), jnp.float32),
            pltpu.VMEM((CP,), jnp.float32),
            pltpu.VMEM((FB + 16,), jnp.int32),
            pltpu.VMEM((FB + 16,), jnp.int32),
            pltpu.VMEM((FB + 16,), jnp.float32),
            pltpu.VMEM((320,), jnp.float32),
            pltpu.VMEM((16,), jnp.int32),
            pltpu.SemaphoreType.DMA,
            pltpu.SemaphoreType.DMA,
        ],
    )
    return f(src, dst, ew)


def _gcn_body(g_h, srcb_h, dstb_h, ewb_h, cnt_h,
              agg_h,
              src0, src1, dst0, dst1, ew0, ew1, rows0, rows1, acc, cbuf,
              sm0, sm1, sr0, sr1):
    t = _wid()
    lo = t * TPN
    row = t * WB
    pltpu.sync_copy(cnt_h.at[pl.ds(_al(16 * t), 16)], cbuf)
    cnt = cbuf[...][0]
    nchunks = (cnt + CE - 1) // CE
    srcs, dsts, ews = (src0, src1), (dst0, dst1), (ew0, ew1)
    rows, sms, srs = (rows0, rows1), (sm0, sm1), (sr0, sr1)

    _zero_vmem(acc, TPN * D)

    def _issue_meta(i, b):
        base = _al(row + i * CE)
        pltpu.async_copy(srcb_h.at[pl.ds(base, CE)], srcs[b], sms[b])
        pltpu.async_copy(dstb_h.at[pl.ds(base, CE)], dsts[b].at[pl.ds(0, CE)], sms[b])
        pltpu.async_copy(ewb_h.at[pl.ds(base, CE)], ews[b].at[pl.ds(0, CE)], sms[b])

    def _wait_meta(i, b):
        base = _al(row + i * CE)
        pltpu.make_async_copy(srcb_h.at[pl.ds(base, CE)], srcs[b], sms[b]).wait()
        pltpu.make_async_copy(dstb_h.at[pl.ds(base, CE)], dsts[b].at[pl.ds(0, CE)], sms[b]).wait()
        pltpu.make_async_copy(ewb_h.at[pl.ds(base, CE)], ews[b].at[pl.ds(0, CE)], sms[b]).wait()

    @pl.when(nchunks > 0)
    def _():
        _issue_meta(0, 0)
        _wait_meta(0, 0)
        pltpu.async_copy(g_h.at[srcs[0]], rows[0], srs[0])

    @pl.when(nchunks > 1)
    def _():
        _issue_meta(1, 1)

    def super_it(i2, _):
        for b in (0, 1):
            i = i2 * 2 + b
            nb = 1 - b

            @pl.when(i < nchunks)
            def _():
                @pl.when(i + 1 < nchunks)
                def _():
                    _wait_meta(i + 1, nb)
                    pltpu.async_copy(g_h.at[srcs[nb]], rows[nb], srs[nb])

                pltpu.make_async_copy(g_h.at[srcs[b]], rows[b], srs[b]).wait()
                base = i * CE
                nin = jnp.minimum(cnt - base, CE)

                def edge(e, _):
                    dl = _sload(dsts[b], e) - lo
                    w = _sload(ews[b], e)
                    a = dl * D
                    for k in range(8):
                        plsc.addupdate(acc.at[pl.ds(a + k * 16, 16)],
                                       w * rows[b][e, pl.ds(k * 16, 16)])
                    return 0
                lax.fori_loop(0, nin, edge, 0)

                @pl.when(i + 2 < nchunks)
                def _():
                    _issue_meta(i + 2, b)
        return 0

    lax.fori_loop(0, (nchunks + 1) // 2, super_it, 0)
    pltpu.sync_copy(acc, agg_h.at[pl.ds(_al(lo * D), TPN * D)])


def _k_gcn(g, srcb, dstb, ewb, cnts):
    f = pl.kernel(
        _gcn_body,
        out_type=[jax.ShapeDtypeStruct((NPAD * D,), jnp.float32)],
        mesh=_mesh(),
        compiler_params=_params(),
        scratch_types=[
            pltpu.VMEM((CE,), jnp.int32),
            pltpu.VMEM((CE,), jnp.int32),
            pltpu.VMEM((CE + 16,), jnp.int32),
            pltpu.VMEM((CE + 16,), jnp.int32),
            pltpu.VMEM((CE + 16,), jnp.float32),
            pltpu.VMEM((CE + 16,), jnp.float32),
            pltpu.VMEM((CE, D), jnp.float32),
            pltpu.VMEM((CE, D), jnp.float32),
            pltpu.VMEM((TPN * D,), jnp.float32),
            pltpu.VMEM((16,), jnp.int32),
            pltpu.SemaphoreType.DMA,
            pltpu.SemaphoreType.DMA,
            pltpu.SemaphoreType.DMA,
            pltpu.SemaphoreType.DMA,
        ],
    )
    return f(g, srcb, dstb, ewb, cnts)[0]


def _gat_body(asm_h, adm_h, asl_h, adl_h, srcb_h, dstb_h, cnt_h,
              emaxm_h, emaxl_h, denm_h, denl_h, exmb_h, exlb_h,
              asm_v, adm_v, asl_v, adl_v,
              emaxm, emaxl, denm, denl,
              src_c, dst_c, exm_c, exl_c, cbuf):
    t = _wid()
    lo = t * TPN
    row = t * WB
    pltpu.sync_copy(cnt_h.at[pl.ds(_al(16 * t), 16)], cbuf)
    cnt = cbuf[...][0]
    pltpu.sync_copy(asm_h, asm_v)
    pltpu.sync_copy(adm_h, adm_v)
    pltpu.sync_copy(asl_h, asl_v)
    pltpu.sync_copy(adl_h, adl_v)
    _fill_vmem(emaxm, 320, NEG)
    _fill_vmem(emaxl, 320, NEG)
    _zero_vmem(denm, 320)
    _zero_vmem(denl, 320)

    nchunks = (cnt + CG - 1) // CG

    def _edge_vals(sv, dv, m):
        svs = jnp.where(m, sv, 0)
        dvs = jnp.where(m, dv, 0)
        em = plsc.load_gather(asm_v, [svs]) + plsc.load_gather(adm_v, [dvs])
        el = plsc.load_gather(asl_v, [svs]) + plsc.load_gather(adl_v, [dvs])
        em = jnp.where(em > 0, em, 0.2 * em)
        el = jnp.where(el > 0, el, 0.2 * el)
        return em, el

    def _seg_max(ref, dloc, ev, m0):
        def again(mm):
            return jnp.any(mm)

        def body(mm):
            cur = plsc.load_gather(ref, [dloc], mask=mm)
            new = jnp.maximum(cur, ev)
            plsc.store_scatter(ref, [dloc], new, mask=mm)
            chk = plsc.load_gather(ref, [dloc], mask=mm)
            return mm & ~(chk >= ev)

        lax.while_loop(again, body, m0)

    def chunkA(ci, _):
        base = ci * CG
        pltpu.sync_copy(srcb_h.at[pl.ds(_al(row + base), CG)], src_c)
        pltpu.sync_copy(dstb_h.at[pl.ds(_al(row + base), CG)], dst_c)
        rem = cnt - base

        def vreg(j, _):
            sv = src_c[pl.ds(j * 16, 16)]
            dv = dst_c[pl.ds(j * 16, 16)]
            m = (lax.iota(jnp.int32, 16) + j * 16) < rem
            em, el = _edge_vals(sv, dv, m)
            dloc = jnp.where(m, dv - lo, 0)
            _seg_max(emaxm, dloc, em, m)
            _seg_max(emaxl, dloc, el, m)
            return 0
        lax.fori_loop(0, CG // 16, vreg, 0)
        return 0
    lax.fori_loop(0, nchunks, chunkA, 0)

    def chunkB(ci, _):
        base = ci * CG
        pltpu.sync_copy(srcb_h.at[pl.ds(_al(row + base), CG)], src_c)
        pltpu.sync_copy(dstb_h.at[pl.ds(_al(row + base), CG)], dst_c)
        rem = cnt - base

        def vreg(j, _):
            sv = src_c[pl.ds(j * 16, 16)]
            dv = dst_c[pl.ds(j * 16, 16)]
            m = (lax.iota(jnp.int32, 16) + j * 16) < rem
            em, el = _edge_vals(sv, dv, m)
            dloc = jnp.where(m, dv - lo, 0)
            exm = jnp.exp(em - plsc.load_gather(emaxm, [dloc]))
            exl = jnp.exp(el - plsc.load_gather(emaxl, [dloc]))
            exm = jnp.where(m, exm, 0.0)
            exl = jnp.where(m, exl, 0.0)
            plsc.addupdate_scatter(denm, [dloc], exm, mask=m)
            plsc.addupdate_scatter(denl, [dloc], exl, mask=m)
            exm_c[pl.ds(j * 16, 16)] = exm
            exl_c[pl.ds(j * 16, 16)] = exl
            return 0
        lax.fori_loop(0, CG // 16, vreg, 0)
        pltpu.sync_copy(exm_c, exmb_h.at[pl.ds(_al(row + base), CG)])
        pltpu.sync_copy(exl_c, exlb_h.at[pl.ds(_al(row + base), CG)])
        return 0
    lax.fori_loop(0, nchunks, chunkB, 0)

    pltpu.sync_copy(emaxm, emaxm_h.at[pl.ds(_al(320 * t), 320)])
    pltpu.sync_copy(emaxl, emaxl_h.at[pl.ds(_al(320 * t), 320)])
    pltpu.sync_copy(denm, denm_h.at[pl.ds(_al(320 * t), 320)])
    pltpu.sync_copy(denl, denl_h.at[pl.ds(_al(320 * t), 320)])


def _k_gat(asm, adm, asl, adl, srcb, dstb, cnts):
    f = pl.kernel(
        _gat_body,
        out_type=[
            jax.ShapeDtypeStruct((NW * 320,), jnp.float32),
            jax.ShapeDtypeStruct((NW * 320,), jnp.float32),
            jax.ShapeDtypeStruct((NW * 320,), jnp.float32),
            jax.ShapeDtypeStruct((NW * 320,), jnp.float32),
            jax.ShapeDtypeStruct((NW * WB,), jnp.float32),
            jax.ShapeDtypeStruct((NW * WB,), jnp.float32),
        ],
        mesh=_mesh(),
        compiler_params=_params(),
        scratch_types=[
            pltpu.VMEM((N,), jnp.float32),
            pltpu.VMEM((N,), jnp.float32),
            pltpu.VMEM((N,), jnp.float32),
            pltpu.VMEM((N,), jnp.float32),
            pltpu.VMEM((320,), jnp.float32),
            pltpu.VMEM((320,), jnp.float32),
            pltpu.VMEM((320,), jnp.float32),
            pltpu.VMEM((320,), jnp.float32),
            pltpu.VMEM((CG,), jnp.int32),
            pltpu.VMEM((CG,), jnp.int32),
            pltpu.VMEM((CG,), jnp.float32),
            pltpu.VMEM((CG,), jnp.float32),
            pltpu.VMEM((16,), jnp.int32),
        ],
    )
    return f(asm, adm, asl, adl, srcb, dstb, cnts)


def _agg_body(h_h, srcb_h, dstb_h, exmb_h, exlb_h, cnt_h,
              nm_h, nl_h,
              src0, src1, dst0, dst1, xm0, xm1, xl0, xl1, rows0, rows1,
              accm, accl, cbuf, sm0, sm1, sr0, sr1):
    t = _wid()
    lo = t * TPN
    row = t * WB
    pltpu.sync_copy(cnt_h.at[pl.ds(_al(16 * t), 16)], cbuf)
    cnt = cbuf[...][0]
    nchunks = (cnt + CE - 1) // CE
    srcs, dsts = (src0, src1), (dst0, dst1)
    xms, xls = (xm0, xm1), (xl0, xl1)
    rows, sms, srs = (rows0, rows1), (sm0, sm1), (sr0, sr1)

    _zero_vmem(accm, TPN * D)
    _zero_vmem(accl, TPN * D)

    def _issue_meta(i, b):
        base = _al(row + i * CE)
        pltpu.async_copy(srcb_h.at[pl.ds(base, CE)], srcs[b], sms[b])
        pltpu.async_copy(dstb_h.at[pl.ds(base, CE)], dsts[b].at[pl.ds(0, CE)], sms[b])
        pltpu.async_copy(exmb_h.at[pl.ds(base, CE)], xms[b].at[pl.ds(0, CE)], sms[b])
        pltpu.async_copy(exlb_h.at[pl.ds(base, CE)], xls[b].at[pl.ds(0, CE)], sms[b])

    def _wait_meta(i, b):
        base = _al(row + i * CE)
        pltpu.make_async_copy(srcb_h.at[pl.ds(base, CE)], srcs[b], sms[b]).wait()
        pltpu.make_async_copy(dstb_h.at[pl.ds(base, CE)], dsts[b].at[pl.ds(0, CE)], sms[b]).wait()
        pltpu.make_async_copy(exmb_h.at[pl.ds(base, CE)], xms[b].at[pl.ds(0, CE)], sms[b]).wait()
        pltpu.make_async_copy(exlb_h.at[pl.ds(base, CE)], xls[b].at[pl.ds(0, CE)], sms[b]).wait()

    @pl.when(nchunks > 0)
    def _():
        _issue_meta(0, 0)
        _wait_meta(0, 0)
        pltpu.async_copy(h_h.at[srcs[0]], rows[0], srs[0])

    @pl.when(nchunks > 1)
    def _():
        _issue_meta(1, 1)

    def super_it(i2, _):
        for b in (0, 1):
            i = i2 * 2 + b
            nb = 1 - b

            @pl.when(i < nchunks)
            def _():
                @pl.when(i + 1 < nchunks)
                def _():
                    _wait_meta(i + 1, nb)
                    pltpu.async_copy(h_h.at[srcs[nb]], rows[nb], srs[nb])

                pltpu.make_async_copy(h_h.at[srcs[b]], rows[b], srs[b]).wait()
                base = i * CE
                nin = jnp.minimum(cnt - base, CE)

                def edge(e, _):
                    dl = _sload(dsts[b], e) - lo
                    wm = _sload(xms[b], e)
                    wl = _sload(xls[b], e)
                    a = dl * D
                    for k in range(8):
                        r = rows[b][e, pl.ds(k * 16, 16)]
                        plsc.addupdate(accm.at[pl.ds(a + k * 16, 16)], wm * r)
                        plsc.addupdate(accl.at[pl.ds(a + k * 16, 16)], wl * r)
                    return 0
                lax.fori_loop(0, nin, edge, 0)

                @pl.when(i + 2 < nchunks)
                def _():
                    _issue_meta(i + 2, b)
        return 0

    lax.fori_loop(0, (nchunks + 1) // 2, super_it, 0)
    pltpu.sync_copy(accm, nm_h.at[pl.ds(_al(lo * D), TPN * D)])
    pltpu.sync_copy(accl, nl_h.at[pl.ds(_al(lo * D), TPN * D)])


def _k_agg(h, srcb, dstb, exmb, exlb, cnts):
    f = pl.kernel(
        _agg_body,
        out_type=[
            jax.ShapeDtypeStruct((NPAD * D,), jnp.float32),
            jax.ShapeDtypeStruct((NPAD * D,), jnp.float32),
        ],
        mesh=_mesh(),
        compiler_params=_params(),
        scratch_types=[
            pltpu.VMEM((CE,), jnp.int32),
            pltpu.VMEM((CE,), jnp.int32),
            pltpu.VMEM((CE + 16,), jnp.int32),
            pltpu.VMEM((CE + 16,), jnp.int32),
            pltpu.VMEM((CE + 16,), jnp.float32),
            pltpu.VMEM((CE + 16,), jnp.float32),
            pltpu.VMEM((CE + 16,), jnp.float32),
            pltpu.VMEM((CE + 16,), jnp.float32),
            pltpu.VMEM((CE, D), jnp.float32),
            pltpu.VMEM((CE, D), jnp.float32),
            pltpu.VMEM((TPN * D,), jnp.float32),
            pltpu.VMEM((TPN * D,), jnp.float32),
            pltpu.VMEM((16,), jnp.int32),
            pltpu.SemaphoreType.DMA,
            pltpu.SemaphoreType.DMA,
            pltpu.SemaphoreType.DMA,
            pltpu.SemaphoreType.DMA,
        ],
    )
    return f(h, srcb, dstb, exmb, exlb, cnts)


RB = 2000


def _rb(shape_cols):
    return pl.BlockSpec((RB, shape_cols), lambda i: (i, 0))


def _full(r, c):
    return pl.BlockSpec((r, c), lambda i: (0, 0))


def _t1_body(x_ref, w_ref, deg_ref, h1_ref, g_ref):
    h1 = jnp.dot(x_ref[...], w_ref[...], preferred_element_type=jnp.float32)
    dinv = lax.rsqrt(deg_ref[...] + 1.0)
    h1_ref[...] = h1
    g_ref[...] = dinv * h1


def _t1(x, W_gcn, deg):
    return pl.pallas_call(
        _t1_body,
        grid=(N // RB,),
        in_specs=[_rb(D), _full(D, D), _rb(1)],
        out_specs=[_rb(D), _rb(D)],
        out_shape=[
            jax.ShapeDtypeStruct((N, D), jnp.float32),
            jax.ShapeDtypeStruct((N, D), jnp.float32),
        ],
    )(x, W_gcn, deg)


def _t3_body(agg_ref, h1_ref, deg_ref, b_ref, wm_ref, am_s_ref, am_d_ref,
             wl_ref, al_s_ref, al_d_ref,
             h_ref, sm_s_ref, sm_d_ref, sl_s_ref, sl_d_ref):
    dinv = lax.rsqrt(deg_ref[...] + 1.0)
    out1 = dinv * agg_ref[...] + (dinv * dinv) * h1_ref[...] + b_ref[...]
    h = jnp.maximum(out1, 0.0)
    h_ref[...] = h
    vm_s = jnp.dot(wm_ref[...], am_s_ref[...], preferred_element_type=jnp.float32)
    vm_d = jnp.dot(wm_ref[...], am_d_ref[...], preferred_element_type=jnp.float32)
    vl_s = jnp.dot(wl_ref[...], al_s_ref[...], preferred_element_type=jnp.float32)
    vl_d = jnp.dot(wl_ref[...], al_d_ref[...], preferred_element_type=jnp.float32)
    sm_s_ref[...] = jnp.dot(h, vm_s, preferred_element_type=jnp.float32)
    sm_d_ref[...] = jnp.dot(h, vm_d, preferred_element_type=jnp.float32)
    sl_s_ref[...] = jnp.dot(h, vl_s, preferred_element_type=jnp.float32)
    sl_d_ref[...] = jnp.dot(h, vl_d, preferred_element_type=jnp.float32)


def _t3(agg, h1, deg, b_gcn, W_mean, a_src_mean, a_dst_mean, W_log,
        a_src_log, a_dst_log):
    return pl.pallas_call(
        _t3_body,
        grid=(N // RB,),
        in_specs=[_rb(D), _rb(D), _rb(1), _full(1, D), _full(D, D),
                  _full(D, 1), _full(D, 1), _full(D, D), _full(D, 1),
                  _full(D, 1)],
        out_specs=[_rb(D), _rb(1), _rb(1), _rb(1), _rb(1)],
        out_shape=[
            jax.ShapeDtypeStruct((N, D), jnp.float32),
            jax.ShapeDtypeStruct((N, 1), jnp.float32),
            jax.ShapeDtypeStruct((N, 1), jnp.float32),
            jax.ShapeDtypeStruct((N, 1), jnp.float32),
            jax.ShapeDtypeStruct((N, 1), jnp.float32),
        ],
    )(agg, h1, deg, b_gcn.reshape(1, D), W_mean, a_src_mean.reshape(D, 1),
      a_dst_mean.reshape(D, 1), W_log, a_src_log.reshape(D, 1),
      a_dst_log.reshape(D, 1))


def _t4_body(nm_ref, nl_ref, h_ref,
             emaxm_ref, denm_ref, emaxl_ref, denl_ref,
             sm_s_ref, sm_d_ref, sl_s_ref, sl_d_ref,
             wm_ref, bm_ref, wl_ref, bl_ref,
             mean_ref, log_ref):
    h = h_ref[...]

    def head(n_agg, emax, den, s_s, s_d, w, b):
        e_self = s_s + s_d
        e_self = jnp.where(e_self > 0, e_self, 0.2 * e_self)
        emax_f = jnp.maximum(emax, e_self)
        r = jnp.exp(emax - emax_f)
        es = jnp.exp(e_self - emax_f)
        dn = r * den + es
        agg = (r * n_agg + es * h) / dn
        return jnp.dot(agg, w, preferred_element_type=jnp.float32) + b

    mean_ref[...] = head(nm_ref[...], emaxm_ref[...], denm_ref[...],
                         sm_s_ref[...], sm_d_ref[...], wm_ref[...], bm_ref[...])
    log_ref[...] = head(nl_ref[...], emaxl_ref[...], denl_ref[...],
                        sl_s_ref[...], sl_d_ref[...], wl_ref[...], bl_ref[...])


def _t4(nm, nl, h, emaxm, denm, emaxl, denl, sm_s, sm_d, sl_s, sl_d,
        W_mean, b_mean, W_log, b_log):
    return pl.pallas_call(
        _t4_body,
        grid=(N // RB,),
        in_specs=[_rb(D), _rb(D), _rb(D),
                  _rb(1), _rb(1), _rb(1), _rb(1),
                  _rb(1), _rb(1), _rb(1), _rb(1),
                  _full(D, D), _full(1, D), _full(D, D), _full(1, D)],
        out_specs=[_rb(D), _rb(D)],
        out_shape=[
            jax.ShapeDtypeStruct((N, D), jnp.float32),
            jax.ShapeDtypeStruct((N, D), jnp.float32),
        ],
    )(nm, nl, h, emaxm, denm, emaxl, denl, sm_s, sm_d, sl_s, sl_d,
      W_mean, b_mean.reshape(1, D), W_log, b_log.reshape(1, D))


def _unpad_nodes(flat):
    return flat.reshape(NW, 320)[:, :TPN].reshape(NPAD)[:N].reshape(N, 1)


def kernel(x, edge_index, edge_weight, W_gcn, b_gcn, W_mean, a_src_mean,
           a_dst_mean, b_mean, W_log, a_src_log, a_dst_log, b_log):
    src = edge_index[0].astype(jnp.int32)
    dst = edge_index[1].astype(jnp.int32)
    ew = edge_weight

    srcb, dstb, ewb, cnts, degp = _k_plan(src, dst, ew)
    deg = _unpad_nodes(degp)

    h1, g = _t1(x, W_gcn, deg)
    aggp = _k_gcn(g, srcb, dstb, ewb, cnts)
    agg = aggp.reshape(NPAD, D)[:N]

    h, sm_s, sm_d, sl_s, sl_d = _t3(agg, h1, deg, b_gcn, W_mean, a_src_mean,
                                    a_dst_mean, W_log, a_src_log, a_dst_log)

    emaxmp, emaxlp, denmp, denlp, exmb, exlb = _k_gat(
        sm_s.reshape(N), sm_d.reshape(N), sl_s.reshape(N), sl_d.reshape(N),
        srcb, dstb, cnts)

    nmp, nlp = _k_agg(h, srcb, dstb, exmb, exlb, cnts)

    mean, logstd = _t4(
        nmp.reshape(NPAD, D)[:N], nlp.reshape(NPAD, D)[:N], h,
        _unpad_nodes(emaxmp), _unpad_nodes(denmp),
        _unpad_nodes(emaxlp), _unpad_nodes(denlp),
        sm_s, sm_d, sl_s, sl_d, W_mean, b_mean, W_log, b_log)
    return (mean, logstd)
